# baseline (device time: 4249813 ns/iter reference)
import jax
import jax.numpy as jnp
from jax import lax
from jax.experimental import pallas as pl
from jax.experimental.pallas import tpu as pltpu

C = 16


def kernel(x):
    m, n = x.shape
    half = m // 2
    ch = half // C

    def body(x_hbm, out_hbm, local_sem, ysend, yrecv, xsend, xrecv):
        my_x = lax.axis_index("x")
        my_y = lax.axis_index("y")
        other_y = 1 - my_y
        other_x = 1 - my_x

        barrier = pltpu.get_barrier_semaphore()
        for dev in [(my_x, other_y), (other_x, my_y)]:
            pl.semaphore_signal(
                barrier, inc=1,
                device_id=dev, device_id_type=pl.DeviceIdType.MESH,
            )
        pl.semaphore_wait(barrier, 2)

        local = pltpu.make_async_copy(
            x_hbm, out_hbm.at[pl.ds(my_y * m, m), :], local_sem
        )
        local.start()

        ydmas = []
        for c in range(C):
            off = my_x * half + c * ch
            ydma = pltpu.make_async_remote_copy(
                src_ref=x_hbm.at[pl.ds(off, ch), :],
                dst_ref=out_hbm.at[pl.ds(my_y * m + off, ch), :],
                send_sem=ysend.at[c],
                recv_sem=yrecv.at[c],
                device_id=(my_x, other_y),
                device_id_type=pl.DeviceIdType.MESH,
            )
            ydma.start()
            ydmas.append(ydma)

        xdmas = []
        for c in range(C):
            ydmas[c].wait_recv()
            roff = other_y * m + my_x * half + c * ch
            xdma = pltpu.make_async_remote_copy(
                src_ref=out_hbm.at[pl.ds(roff, ch), :],
                dst_ref=out_hbm.at[pl.ds(roff, ch), :],
                send_sem=xsend.at[c],
                recv_sem=xrecv.at[c],
                device_id=(other_x, my_y),
                device_id_type=pl.DeviceIdType.MESH,
            )
            xdma.start()
            xdmas.append(xdma)

        for c in range(C):
            xdmas[c].wait_recv()
        for c in range(C):
            ydmas[c].wait_send()
            xdmas[c].wait_send()
        local.wait()

    return pl.pallas_call(
        body,
        out_shape=jax.ShapeDtypeStruct((2 * m, n), x.dtype),
        in_specs=[pl.BlockSpec(memory_space=pl.ANY)],
        out_specs=pl.BlockSpec(memory_space=pl.ANY),
        scratch_shapes=[
            pltpu.SemaphoreType.DMA,
            pltpu.SemaphoreType.DMA((C,)),
            pltpu.SemaphoreType.DMA((C,)),
            pltpu.SemaphoreType.DMA((C,)),
            pltpu.SemaphoreType.DMA((C,)),
        ],
        compiler_params=pltpu.CompilerParams(collective_id=0),
    )(x)


# device time: 1002571 ns/iter; 4.2389x vs baseline; 4.2389x over previous
import jax
import jax.numpy as jnp
from jax import lax
from jax.experimental import pallas as pl
from jax.experimental.pallas import tpu as pltpu

C = 32
LC = 16


def kernel(x):
    m, n = x.shape
    half = m // 2
    ch = half // C
    lch = m // LC

    def body(x_hbm, out_hbm, stage, isems, osems, ysend, yrecv, xsend, xrecv):
        my_x = lax.axis_index("x")
        my_y = lax.axis_index("y")
        other_y = 1 - my_y
        other_x = 1 - my_x

        barrier = pltpu.get_barrier_semaphore()
        for dev in [(my_x, other_y), (other_x, my_y)]:
            pl.semaphore_signal(
                barrier, inc=1,
                device_id=dev, device_id_type=pl.DeviceIdType.MESH,
            )
        pl.semaphore_wait(barrier, 2)

        ydmas = []
        for c in range(C):
            off = my_x * half + c * ch
            ydma = pltpu.make_async_remote_copy(
                src_ref=x_hbm.at[pl.ds(off, ch), :],
                dst_ref=out_hbm.at[pl.ds(my_y * m + off, ch), :],
                send_sem=ysend.at[c],
                recv_sem=yrecv.at[c],
                device_id=(my_x, other_y),
                device_id_type=pl.DeviceIdType.MESH,
            )
            ydma.start()
            ydmas.append(ydma)

        stores = [None] * LC
        for c in range(LC):
            slot = c % 2
            if c >= 2:
                stores[c - 2].wait()
            ld = pltpu.make_async_copy(
                x_hbm.at[pl.ds(c * lch, lch), :], stage.at[slot], isems.at[slot]
            )
            ld.start()
            ld.wait()
            st = pltpu.make_async_copy(
                stage.at[slot],
                out_hbm.at[pl.ds(my_y * m + c * lch, lch), :],
                osems.at[slot],
            )
            st.start()
            stores[c] = st
        stores[LC - 2].wait()
        stores[LC - 1].wait()

        xdmas = []
        for c in range(C):
            ydmas[c].wait_recv()
            roff = other_y * m + my_x * half + c * ch
            xdma = pltpu.make_async_remote_copy(
                src_ref=out_hbm.at[pl.ds(roff, ch), :],
                dst_ref=out_hbm.at[pl.ds(roff, ch), :],
                send_sem=xsend.at[c],
                recv_sem=xrecv.at[c],
                device_id=(other_x, my_y),
                device_id_type=pl.DeviceIdType.MESH,
            )
            xdma.start()
            xdmas.append(xdma)

        for c in range(C):
            xdmas[c].wait_recv()
        for c in range(C):
            ydmas[c].wait_send()
            xdmas[c].wait_send()

    return pl.pallas_call(
        body,
        out_shape=jax.ShapeDtypeStruct((2 * m, n), x.dtype),
        in_specs=[pl.BlockSpec(memory_space=pl.ANY)],
        out_specs=pl.BlockSpec(memory_space=pl.ANY),
        scratch_shapes=[
            pltpu.VMEM((2, m // LC, n), jnp.float32),
            pltpu.SemaphoreType.DMA((2,)),
            pltpu.SemaphoreType.DMA((2,)),
            pltpu.SemaphoreType.DMA((C,)),
            pltpu.SemaphoreType.DMA((C,)),
            pltpu.SemaphoreType.DMA((C,)),
            pltpu.SemaphoreType.DMA((C,)),
        ],
        compiler_params=pltpu.CompilerParams(collective_id=0),
    )(x)


# device time: 920031 ns/iter; 4.6192x vs baseline; 1.0897x over previous
import jax
import jax.numpy as jnp
from jax import lax
from jax.experimental import pallas as pl
from jax.experimental.pallas import tpu as pltpu

C = 32
LC = 16


def kernel(x):
    m, n = x.shape
    half = m // 2
    ch = half // C
    lch = m // LC

    def body(x_hbm, out_hbm, stage, isems, osems, ysend, yrecv, xsend, xrecv):
        my_x = lax.axis_index("x")
        my_y = lax.axis_index("y")
        other_y = 1 - my_y
        other_x = 1 - my_x

        barrier = pltpu.get_barrier_semaphore()
        for dev in [(my_x, other_y), (other_x, my_y)]:
            pl.semaphore_signal(
                barrier, inc=1,
                device_id=dev, device_id_type=pl.DeviceIdType.MESH,
            )
        pl.semaphore_wait(barrier, 2)

        ydmas = []
        for c in range(C):
            off = my_x * half + c * ch
            ydma = pltpu.make_async_remote_copy(
                src_ref=x_hbm.at[pl.ds(off, ch), :],
                dst_ref=out_hbm.at[pl.ds(my_y * m + off, ch), :],
                send_sem=ysend.at[c],
                recv_sem=yrecv.at[c],
                device_id=(my_x, other_y),
                device_id_type=pl.DeviceIdType.MESH,
            )
            ydma.start()
            ydmas.append(ydma)

        stores = [None] * LC

        def stage_chunk(c):
            slot = c % 2
            if c >= 2:
                stores[c - 2].wait()
            ld = pltpu.make_async_copy(
                x_hbm.at[pl.ds(c * lch, lch), :], stage.at[slot], isems.at[slot]
            )
            ld.start()
            ld.wait()
            st = pltpu.make_async_copy(
                stage.at[slot],
                out_hbm.at[pl.ds(my_y * m + c * lch, lch), :],
                osems.at[slot],
            )
            st.start()
            stores[c] = st

        xdmas = []
        for c in range(C):
            ydmas[c].wait_recv()
            roff = other_y * m + my_x * half + c * ch
            xdma = pltpu.make_async_remote_copy(
                src_ref=out_hbm.at[pl.ds(roff, ch), :],
                dst_ref=out_hbm.at[pl.ds(roff, ch), :],
                send_sem=xsend.at[c],
                recv_sem=xrecv.at[c],
                device_id=(other_x, my_y),
                device_id_type=pl.DeviceIdType.MESH,
            )
            xdma.start()
            xdmas.append(xdma)
            if c < LC:
                stage_chunk(c)
        stores[LC - 2].wait()
        stores[LC - 1].wait()

        for c in range(C):
            xdmas[c].wait_recv()
        for c in range(C):
            ydmas[c].wait_send()
            xdmas[c].wait_send()

    return pl.pallas_call(
        body,
        out_shape=jax.ShapeDtypeStruct((2 * m, n), x.dtype),
        in_specs=[pl.BlockSpec(memory_space=pl.ANY)],
        out_specs=pl.BlockSpec(memory_space=pl.ANY),
        scratch_shapes=[
            pltpu.VMEM((2, m // LC, n), jnp.float32),
            pltpu.SemaphoreType.DMA((2,)),
            pltpu.SemaphoreType.DMA((2,)),
            pltpu.SemaphoreType.DMA((C,)),
            pltpu.SemaphoreType.DMA((C,)),
            pltpu.SemaphoreType.DMA((C,)),
            pltpu.SemaphoreType.DMA((C,)),
        ],
        compiler_params=pltpu.CompilerParams(collective_id=0),
    )(x)
